# Initial kernel scaffold; baseline (speedup 1.0000x reference)
import functools

import jax
import jax.numpy as jnp
from jax import lax
from jax.experimental import pallas as pl
from jax.experimental.pallas import tpu as pltpu
from jax.experimental.pallas import tpu_sc as plsc

N = 10000
D = 128
E = 320000
W = 144
NC, NS = 2, 16
EDGES_PER_TILE = E // (NC * NS)
CHUNK = 80
NCHUNKS = EDGES_PER_TILE // CHUNK
ROWS_PER_TILE = N // NS
ZCOPIES = ROWS_PER_TILE // CHUNK


def _sc_aggregate(xa, src, dst):
    mesh = plsc.VectorSubcoreMesh(core_axis_name="c", subcore_axis_name="s")

    @functools.partial(
        pl.kernel,
        out_type=jax.ShapeDtypeStruct((NC, N, W), jnp.float32),
        mesh=mesh,
        scratch_types=[
            pltpu.VMEM((CHUNK,), jnp.int32),
            pltpu.VMEM((CHUNK,), jnp.int32),
            pltpu.VMEM((CHUNK, W), jnp.float32),
            pltpu.VMEM_SHARED((N, W), jnp.float32),
        ],
    )
    def agg_kernel(xa_hbm, src_hbm, dst_hbm, out_hbm, sidx, didx, rows, acc):
        cid = lax.axis_index("c")
        sid = lax.axis_index("s")

        @pl.loop(0, CHUNK)
        def _(i):
            @pl.loop(0, W // 16)
            def _(j):
                rows[i, pl.ds(j * 16, 16)] = jnp.zeros((16,), jnp.float32)

        row0 = sid * ROWS_PER_TILE

        @pl.loop(0, ZCOPIES)
        def _(r):
            pltpu.sync_copy(rows, acc.at[pl.ds(row0 + r * CHUNK, CHUNK)])

        tail = ROWS_PER_TILE - ZCOPIES * CHUNK
        pltpu.sync_copy(
            rows.at[pl.ds(0, tail)],
            acc.at[pl.ds(row0 + ZCOPIES * CHUNK, tail)],
        )
        plsc.subcore_barrier()

        base = (cid * NS + sid) * EDGES_PER_TILE

        @pl.loop(0, NCHUNKS)
        def _(g):
            off = base + g * CHUNK
            pltpu.sync_copy(src_hbm.at[pl.ds(off, CHUNK)], sidx)
            pltpu.sync_copy(dst_hbm.at[pl.ds(off, CHUNK)], didx)
            pltpu.sync_copy(xa_hbm.at[sidx], rows)
            pltpu.sync_copy(rows, acc.at[didx], add=True)

        plsc.subcore_barrier()
        pltpu.sync_copy(
            acc.at[pl.ds(row0, ROWS_PER_TILE)],
            out_hbm.at[cid, pl.ds(row0, ROWS_PER_TILE)],
        )

    return agg_kernel(xa, src, dst)


BN = 1000


def _tc_linear_r(xa, Wr):
    def body(x_ref, w_ref, o_ref):
        o_ref[...] = lax.dot_general(
            x_ref[:, :D],
            w_ref[...],
            (((1,), (1,)), ((), ())),
            preferred_element_type=jnp.float32,
        )

    return pl.pallas_call(
        body,
        grid=(N // BN,),
        in_specs=[
            pl.BlockSpec((BN, W), lambda i: (i, 0)),
            pl.BlockSpec((D, D), lambda i: (0, 0)),
        ],
        out_specs=pl.BlockSpec((BN, D), lambda i: (i, 0)),
        out_shape=jax.ShapeDtypeStruct((N, D), jnp.float32),
    )(xa, Wr)


def _tc_combine(p, r, Wl, bl, aug):
    OW = W if aug else D

    def body(p_ref, r_ref, w_ref, b_ref, o_ref):
        s = p_ref[0] + p_ref[1]
        cnt = s[:, D : D + 1]
        mean = s[:, :D] / jnp.maximum(cnt, 1.0)
        out = lax.dot_general(
            mean,
            w_ref[...],
            (((1,), (1,)), ((), ())),
            preferred_element_type=jnp.float32,
        )
        out = jnp.maximum(out + b_ref[...] + r_ref[...], 0.0)
        if aug:
            padcol = (
                lax.broadcasted_iota(jnp.int32, (BN, W - D), 1) == 0
            ).astype(jnp.float32)
            o_ref[...] = jnp.concatenate([out, padcol], axis=1)
        else:
            o_ref[...] = out

    return pl.pallas_call(
        body,
        grid=(N // BN,),
        in_specs=[
            pl.BlockSpec((NC, BN, W), lambda i: (0, i, 0)),
            pl.BlockSpec((BN, D), lambda i: (i, 0)),
            pl.BlockSpec((D, D), lambda i: (0, 0)),
            pl.BlockSpec((1, D), lambda i: (0, 0)),
        ],
        out_specs=pl.BlockSpec((BN, OW), lambda i: (i, 0)),
        out_shape=jax.ShapeDtypeStruct((N, OW), jnp.float32),
    )(p, r, Wl, bl.reshape(1, D))


def kernel(x, edge_index, W_l1, b_l1, W_r1, W_l2, b_l2, W_r2):
    xa = jnp.concatenate(
        [x, jnp.ones((N, 1), x.dtype), jnp.zeros((N, W - D - 1), x.dtype)], axis=1
    )
    src = edge_index[0]
    dst = edge_index[1]

    p1 = _sc_aggregate(xa, src, dst)
    r1 = _tc_linear_r(xa, W_r1)
    ha = _tc_combine(p1, r1, W_l1, b_l1, aug=True)

    p2 = _sc_aggregate(ha, src, dst)
    r2 = _tc_linear_r(ha, W_r2)
    return _tc_combine(p2, r2, W_l2, b_l2, aug=False)


# speedup vs baseline: 4.5682x; 4.5682x over previous
import functools

import jax
import jax.numpy as jnp
from jax import lax
from jax.experimental import pallas as pl
from jax.experimental.pallas import tpu as pltpu
from jax.experimental.pallas import tpu_sc as plsc

N = 10000
D = 128
E = 320000
W = 144
NC, NS = 2, 16
EDGES_PER_TILE = E // (NC * NS)
CHUNK = 80
NCHUNKS = EDGES_PER_TILE // CHUNK
NP = 10240
ROWS_PER_TILE = NP // NS
ZCOPIES = ROWS_PER_TILE // CHUNK


def _sc_aggregate(xa, src, dst):
    mesh = plsc.VectorSubcoreMesh(core_axis_name="c", subcore_axis_name="s")

    @functools.partial(
        pl.kernel,
        out_type=jax.ShapeDtypeStruct((NC, NP, W), jnp.float32),
        mesh=mesh,
        scratch_types=[
            pltpu.VMEM((CHUNK,), jnp.int32),
            pltpu.VMEM((CHUNK,), jnp.int32),
            pltpu.VMEM((CHUNK, W), jnp.float32),
            pltpu.VMEM_SHARED((NP, W), jnp.float32),
        ],
        compiler_params=pltpu.CompilerParams(use_tc_tiling_on_sc=False),
    )
    def agg_kernel(xa_hbm, src_hbm, dst_hbm, out_hbm, sidx, didx, rows, acc):
        cid = lax.axis_index("c")
        sid = lax.axis_index("s")

        @pl.loop(0, CHUNK)
        def _(i):
            @pl.loop(0, W // 16)
            def _(j):
                rows[i, pl.ds(j * 16, 16)] = jnp.zeros((16,), jnp.float32)

        row0 = sid * ROWS_PER_TILE

        @pl.loop(0, ZCOPIES)
        def _(r):
            pltpu.sync_copy(rows, acc.at[pl.ds(row0 + r * CHUNK, CHUNK)])

        plsc.subcore_barrier()

        base = (cid * NS + sid) * EDGES_PER_TILE

        @pl.loop(0, NCHUNKS)
        def _(g):
            off = base + g * CHUNK
            pltpu.sync_copy(src_hbm.at[pl.ds(off, CHUNK)], sidx)
            pltpu.sync_copy(dst_hbm.at[pl.ds(off, CHUNK)], didx)
            pltpu.sync_copy(xa_hbm.at[sidx], rows)
            pltpu.sync_copy(rows, acc.at[didx], add=True)

        plsc.subcore_barrier()
        pltpu.sync_copy(
            acc.at[pl.ds(row0, ROWS_PER_TILE)],
            out_hbm.at[cid, pl.ds(row0, ROWS_PER_TILE)],
        )

    return agg_kernel(xa, src, dst)


BN = 1000


def _tc_linear_r(xa, Wr):
    def body(x_ref, w_ref, o_ref):
        o_ref[...] = lax.dot_general(
            x_ref[:, :D],
            w_ref[...],
            (((1,), (1,)), ((), ())),
            preferred_element_type=jnp.float32,
        )

    return pl.pallas_call(
        body,
        grid=(N // BN,),
        in_specs=[
            pl.BlockSpec((BN, W), lambda i: (i, 0)),
            pl.BlockSpec((D, D), lambda i: (0, 0)),
        ],
        out_specs=pl.BlockSpec((BN, D), lambda i: (i, 0)),
        out_shape=jax.ShapeDtypeStruct((N, D), jnp.float32),
    )(xa, Wr)


def _tc_combine(p, r, Wl, bl, aug):
    OW = W if aug else D

    def body(p_ref, r_ref, w_ref, b_ref, o_ref):
        s = p_ref[0] + p_ref[1]
        cnt = s[:, D : D + 1]
        mean = s[:, :D] / jnp.maximum(cnt, 1.0)
        out = lax.dot_general(
            mean,
            w_ref[...],
            (((1,), (1,)), ((), ())),
            preferred_element_type=jnp.float32,
        )
        out = jnp.maximum(out + b_ref[...] + r_ref[...], 0.0)
        if aug:
            padcol = (
                lax.broadcasted_iota(jnp.int32, (BN, W - D), 1) == 0
            ).astype(jnp.float32)
            o_ref[...] = jnp.concatenate([out, padcol], axis=1)
        else:
            o_ref[...] = out

    return pl.pallas_call(
        body,
        grid=(N // BN,),
        in_specs=[
            pl.BlockSpec((NC, BN, W), lambda i: (0, i, 0)),
            pl.BlockSpec((BN, D), lambda i: (i, 0)),
            pl.BlockSpec((D, D), lambda i: (0, 0)),
            pl.BlockSpec((1, D), lambda i: (0, 0)),
        ],
        out_specs=pl.BlockSpec((BN, OW), lambda i: (i, 0)),
        out_shape=jax.ShapeDtypeStruct((N, OW), jnp.float32),
    )(p, r, Wl, bl.reshape(1, D))


def kernel(x, edge_index, W_l1, b_l1, W_r1, W_l2, b_l2, W_r2):
    xa = jnp.concatenate(
        [x, jnp.ones((N, 1), x.dtype), jnp.zeros((N, W - D - 1), x.dtype)], axis=1
    )
    src = edge_index[0]
    dst = edge_index[1]

    p1 = _sc_aggregate(xa, src, dst)
    r1 = _tc_linear_r(xa, W_r1)
    ha = _tc_combine(p1, r1, W_l1, b_l1, aug=True)

    p2 = _sc_aggregate(ha, src, dst)
    r2 = _tc_linear_r(ha, W_r2)
    return _tc_combine(p2, r2, W_l2, b_l2, aug=False)


# speedup vs baseline: 11.6113x; 2.5418x over previous
import functools

import jax
import jax.numpy as jnp
from jax import lax
from jax.experimental import pallas as pl
from jax.experimental.pallas import tpu as pltpu
from jax.experimental.pallas import tpu_sc as plsc

N = 10000
D = 128
E = 320000
CW = 16
NC, NS = 2, 16
EDGES_PER_TILE = E // (NC * NS)
CHUNK = 80
NCHUNKS = EDGES_PER_TILE // CHUNK
NP = 10240
ROWS_PER_TILE = NP // NS
ZCOPIES = ROWS_PER_TILE // CHUNK
NBUF = 2


def _sc_aggregate(xf, pk3, with_counts):
    mesh = plsc.VectorSubcoreMesh(core_axis_name="c", subcore_axis_name="s")

    out_type = [jax.ShapeDtypeStruct((NC, NP, D), jnp.float32)]
    scratch = [
        pltpu.VMEM((EDGES_PER_TILE,), jnp.int32),
        pltpu.VMEM((NBUF, CHUNK), jnp.int32),
        pltpu.VMEM((NBUF, CHUNK), jnp.int32),
        pltpu.VMEM((NBUF, CHUNK, D), jnp.float32),
        pltpu.SemaphoreType.DMA((NBUF,)),
        pltpu.SemaphoreType.DMA((NBUF,)),
    ]
    if with_counts:
        out_type.append(jax.ShapeDtypeStruct((NC, NP, CW), jnp.float32))
        scratch += [
            pltpu.VMEM((CHUNK, CW), jnp.float32),
            pltpu.VMEM((CHUNK, CW), jnp.float32),
            pltpu.SemaphoreType.DMA((NBUF,)),
        ]
        scratch.append(pltpu.VMEM_SHARED((NP, CW), jnp.float32))
    scratch.append(pltpu.VMEM_SHARED((NP, D), jnp.float32))

    @functools.partial(
        pl.kernel,
        out_type=out_type,
        mesh=mesh,
        scratch_types=scratch,
        compiler_params=pltpu.CompilerParams(use_tc_tiling_on_sc=False),
    )
    def agg_kernel(xf_hbm, pk_hbm, *rest):
        if with_counts:
            (outf_hbm, outc_hbm, pkv, sidx, didx, rows, gsem, ssem,
             ones, zbuf, csem, accc, accf) = rest
        else:
            outf_hbm, pkv, sidx, didx, rows, gsem, ssem, accf = rest

        cid = lax.axis_index("c")
        sid = lax.axis_index("s")
        tid = cid * NS + sid

        pltpu.sync_copy(pk_hbm.at[tid], pkv)

        @pl.loop(0, CHUNK)
        def _(i):
            @pl.loop(0, D // 16)
            def _(j):
                rows[0, i, pl.ds(j * 16, 16)] = jnp.zeros((16,), jnp.float32)
            if with_counts:
                ones[i, :] = jnp.ones((CW,), jnp.float32)
                zbuf[i, :] = jnp.zeros((CW,), jnp.float32)

        row0 = sid * ROWS_PER_TILE

        @pl.loop(0, ZCOPIES)
        def _(r):
            pltpu.sync_copy(rows.at[0], accf.at[pl.ds(row0 + r * CHUNK, CHUNK)])

        if with_counts:
            @pl.loop(0, ZCOPIES)
            def _(r):
                pltpu.sync_copy(zbuf, accc.at[pl.ds(row0 + r * CHUNK, CHUNK)])

        plsc.subcore_barrier()

        def unpack(g, b):
            off = g * CHUNK
            for k in range(CHUNK // 16):
                v = pkv[pl.ds(off + k * 16, 16)]
                sidx[b, pl.ds(k * 16, 16)] = lax.bitwise_and(v, 0xFFFF)
                didx[b, pl.ds(k * 16, 16)] = lax.shift_right_logical(v, 16)

        def fire_gather(b):
            pltpu.async_copy(xf_hbm.at[sidx.at[b]], rows.at[b], gsem.at[b])

        def wait_gather(b):
            pltpu.make_async_copy(
                xf_hbm.at[sidx.at[b]], rows.at[b], gsem.at[b]
            ).wait()

        for b in range(NBUF):
            unpack(b, b)
            fire_gather(b)

        def step(g, b):
            wait_gather(b)
            sc = pltpu.async_copy(
                rows.at[b], accf.at[didx.at[b]], ssem.at[b], add=True
            )
            if with_counts:
                cc = pltpu.async_copy(
                    ones, accc.at[didx.at[b]], csem.at[b], add=True
                )
            sc.wait()
            if with_counts:
                cc.wait()
            g2 = g + NBUF

            @pl.when(g2 < NCHUNKS)
            def _():
                unpack(g2, b)
                fire_gather(b)

        @pl.loop(0, NCHUNKS // NBUF)
        def _(i):
            for b in range(NBUF):
                step(i * NBUF + b, b)

        for b in range(NCHUNKS % NBUF):
            step((NCHUNKS // NBUF) * NBUF + b, b)

        plsc.subcore_barrier()
        pltpu.sync_copy(
            accf.at[pl.ds(row0, ROWS_PER_TILE)],
            outf_hbm.at[cid, pl.ds(row0, ROWS_PER_TILE)],
        )
        if with_counts:
            pltpu.sync_copy(
                accc.at[pl.ds(row0, ROWS_PER_TILE)],
                outc_hbm.at[cid, pl.ds(row0, ROWS_PER_TILE)],
            )

    return agg_kernel(xf, pk3)


BN = 1000


def _tc_linear_r(xf, Wr):
    def body(x_ref, w_ref, o_ref):
        o_ref[...] = lax.dot_general(
            x_ref[...],
            w_ref[...],
            (((1,), (1,)), ((), ())),
            preferred_element_type=jnp.float32,
        )

    return pl.pallas_call(
        body,
        grid=(N // BN,),
        in_specs=[
            pl.BlockSpec((BN, D), lambda i: (i, 0)),
            pl.BlockSpec((D, D), lambda i: (0, 0)),
        ],
        out_specs=pl.BlockSpec((BN, D), lambda i: (i, 0)),
        out_shape=jax.ShapeDtypeStruct((N, D), jnp.float32),
    )(xf, Wr)


def _tc_combine(p, co, r, Wl, bl):
    def body(p_ref, c_ref, r_ref, w_ref, b_ref, o_ref):
        s = p_ref[0] + p_ref[1]
        cnt = (c_ref[0] + c_ref[1])[:, 0:1]
        mean = s / jnp.maximum(cnt, 1.0)
        out = lax.dot_general(
            mean,
            w_ref[...],
            (((1,), (1,)), ((), ())),
            preferred_element_type=jnp.float32,
        )
        o_ref[...] = jnp.maximum(out + b_ref[...] + r_ref[...], 0.0)

    return pl.pallas_call(
        body,
        grid=(N // BN,),
        in_specs=[
            pl.BlockSpec((NC, BN, D), lambda i: (0, i, 0)),
            pl.BlockSpec((NC, BN, CW), lambda i: (0, i, 0)),
            pl.BlockSpec((BN, D), lambda i: (i, 0)),
            pl.BlockSpec((D, D), lambda i: (0, 0)),
            pl.BlockSpec((1, D), lambda i: (0, 0)),
        ],
        out_specs=pl.BlockSpec((BN, D), lambda i: (i, 0)),
        out_shape=jax.ShapeDtypeStruct((N, D), jnp.float32),
    )(p, co, r, Wl, bl.reshape(1, D))


def kernel(x, edge_index, W_l1, b_l1, W_r1, W_l2, b_l2, W_r2):
    src = edge_index[0]
    dst = edge_index[1]
    pk3 = (src | (dst << 16)).reshape(NC * NS, EDGES_PER_TILE)

    p1, c1 = _sc_aggregate(x, pk3, with_counts=True)
    r1 = _tc_linear_r(x, W_r1)
    h = _tc_combine(p1, c1, r1, W_l1, b_l1)

    (p2,) = _sc_aggregate(h, pk3, with_counts=False)
    r2 = _tc_linear_r(h, W_r2)
    return _tc_combine(p2, c1, r2, W_l2, b_l2)


# speedup vs baseline: 13.7003x; 1.1799x over previous
import functools

import jax
import jax.numpy as jnp
from jax import lax
from jax.experimental import pallas as pl
from jax.experimental.pallas import tpu as pltpu
from jax.experimental.pallas import tpu_sc as plsc

N = 10000
D = 128
E = 320000
CW = 16
NC, NS = 2, 16
EDGES_PER_TILE = E // (NC * NS)
NP = 10240
ROWS_PER_TILE = NP // NS


def _sc_aggregate(xf, pk3, with_counts, chunk, nbuf):
    CHUNK, NBUF = chunk, nbuf
    NCHUNKS = EDGES_PER_TILE // CHUNK
    ZCOPIES = ROWS_PER_TILE // CHUNK
    mesh = plsc.VectorSubcoreMesh(core_axis_name="c", subcore_axis_name="s")

    out_type = [jax.ShapeDtypeStruct((NC, NP, D), jnp.float32)]
    scratch = [
        pltpu.VMEM((EDGES_PER_TILE,), jnp.int32),
        pltpu.VMEM((NBUF, CHUNK), jnp.int32),
        pltpu.VMEM((NBUF, CHUNK), jnp.int32),
        pltpu.VMEM((NBUF, CHUNK, D), jnp.float32),
        pltpu.SemaphoreType.DMA((NBUF,)),
        pltpu.SemaphoreType.DMA((NBUF,)),
    ]
    if with_counts:
        out_type.append(jax.ShapeDtypeStruct((NC, NP, CW), jnp.float32))
        scratch += [
            pltpu.VMEM((CHUNK, CW), jnp.float32),
            pltpu.VMEM((CHUNK, CW), jnp.float32),
            pltpu.SemaphoreType.DMA((NBUF,)),
        ]
        scratch.append(pltpu.VMEM_SHARED((NP, CW), jnp.float32))
    scratch.append(pltpu.VMEM_SHARED((NP, D), jnp.float32))

    @functools.partial(
        pl.kernel,
        out_type=out_type,
        mesh=mesh,
        scratch_types=scratch,
        compiler_params=pltpu.CompilerParams(use_tc_tiling_on_sc=False),
    )
    def agg_kernel(xf_hbm, pk_hbm, *rest):
        if with_counts:
            (outf_hbm, outc_hbm, pkv, sidx, didx, rows, gsem, ssem,
             ones, zbuf, csem, accc, accf) = rest
        else:
            outf_hbm, pkv, sidx, didx, rows, gsem, ssem, accf = rest

        cid = lax.axis_index("c")
        sid = lax.axis_index("s")
        tid = cid * NS + sid

        pltpu.sync_copy(pk_hbm.at[tid], pkv)

        @pl.loop(0, CHUNK)
        def _(i):
            @pl.loop(0, D // 16)
            def _(j):
                rows[0, i, pl.ds(j * 16, 16)] = jnp.zeros((16,), jnp.float32)
            if with_counts:
                ones[i, :] = jnp.ones((CW,), jnp.float32)
                zbuf[i, :] = jnp.zeros((CW,), jnp.float32)

        row0 = sid * ROWS_PER_TILE

        @pl.loop(0, ZCOPIES)
        def _(r):
            pltpu.sync_copy(rows.at[0], accf.at[pl.ds(row0 + r * CHUNK, CHUNK)])

        if with_counts:
            @pl.loop(0, ZCOPIES)
            def _(r):
                pltpu.sync_copy(zbuf, accc.at[pl.ds(row0 + r * CHUNK, CHUNK)])

        plsc.subcore_barrier()

        ks = list(range(0, CHUNK - 15, 16))
        if CHUNK % 16:
            ks.append(CHUNK - 16)

        def unpack(g, b):
            off = g * CHUNK
            for k in ks:
                v = pkv[pl.ds(off + k, 16)]
                sidx[b, pl.ds(k, 16)] = lax.bitwise_and(v, 0xFFFF)
                didx[b, pl.ds(k, 16)] = lax.shift_right_logical(v, 16)

        def fire_gather(b):
            pltpu.async_copy(xf_hbm.at[sidx.at[b]], rows.at[b], gsem.at[b])

        def wait_gather(b):
            pltpu.make_async_copy(
                xf_hbm.at[sidx.at[b]], rows.at[b], gsem.at[b]
            ).wait()

        for b in range(NBUF):
            unpack(b, b)
            fire_gather(b)

        def step(g, b):
            wait_gather(b)
            sc = pltpu.async_copy(
                rows.at[b], accf.at[didx.at[b]], ssem.at[b], add=True
            )
            if with_counts:
                cc = pltpu.async_copy(
                    ones, accc.at[didx.at[b]], csem.at[b], add=True
                )
            sc.wait()
            if with_counts:
                cc.wait()
            g2 = g + NBUF

            @pl.when(g2 < NCHUNKS)
            def _():
                unpack(g2, b)
                fire_gather(b)

        @pl.loop(0, NCHUNKS // NBUF)
        def _(i):
            for b in range(NBUF):
                step(i * NBUF + b, b)

        for b in range(NCHUNKS % NBUF):
            step((NCHUNKS // NBUF) * NBUF + b, b)

        plsc.subcore_barrier()
        pltpu.sync_copy(
            accf.at[pl.ds(row0, ROWS_PER_TILE)],
            outf_hbm.at[cid, pl.ds(row0, ROWS_PER_TILE)],
        )
        if with_counts:
            pltpu.sync_copy(
                accc.at[pl.ds(row0, ROWS_PER_TILE)],
                outc_hbm.at[cid, pl.ds(row0, ROWS_PER_TILE)],
            )

    return agg_kernel(xf, pk3)


BN = 1000


def _tc_linear_r(xf, Wr):
    def body(x_ref, w_ref, o_ref):
        o_ref[...] = lax.dot_general(
            x_ref[...],
            w_ref[...],
            (((1,), (1,)), ((), ())),
            preferred_element_type=jnp.float32,
        )

    return pl.pallas_call(
        body,
        grid=(N // BN,),
        in_specs=[
            pl.BlockSpec((BN, D), lambda i: (i, 0)),
            pl.BlockSpec((D, D), lambda i: (0, 0)),
        ],
        out_specs=pl.BlockSpec((BN, D), lambda i: (i, 0)),
        out_shape=jax.ShapeDtypeStruct((N, D), jnp.float32),
    )(xf, Wr)


def _tc_combine(p, co, r, Wl, bl):
    def body(p_ref, c_ref, r_ref, w_ref, b_ref, o_ref):
        s = p_ref[0] + p_ref[1]
        cnt = (c_ref[0] + c_ref[1])[:, 0:1]
        mean = s / jnp.maximum(cnt, 1.0)
        out = lax.dot_general(
            mean,
            w_ref[...],
            (((1,), (1,)), ((), ())),
            preferred_element_type=jnp.float32,
        )
        o_ref[...] = jnp.maximum(out + b_ref[...] + r_ref[...], 0.0)

    return pl.pallas_call(
        body,
        grid=(N // BN,),
        in_specs=[
            pl.BlockSpec((NC, BN, D), lambda i: (0, i, 0)),
            pl.BlockSpec((NC, BN, CW), lambda i: (0, i, 0)),
            pl.BlockSpec((BN, D), lambda i: (i, 0)),
            pl.BlockSpec((D, D), lambda i: (0, 0)),
            pl.BlockSpec((1, D), lambda i: (0, 0)),
        ],
        out_specs=pl.BlockSpec((BN, D), lambda i: (i, 0)),
        out_shape=jax.ShapeDtypeStruct((N, D), jnp.float32),
    )(p, co, r, Wl, bl.reshape(1, D))


def kernel(x, edge_index, W_l1, b_l1, W_r1, W_l2, b_l2, W_r2):
    src = edge_index[0]
    dst = edge_index[1]
    pk3 = (src | (dst << 16)).reshape(NC * NS, EDGES_PER_TILE)

    p1, c1 = _sc_aggregate(x, pk3, with_counts=True, chunk=40, nbuf=5)
    r1 = _tc_linear_r(x, W_r1)
    h = _tc_combine(p1, c1, r1, W_l1, b_l1)

    (p2,) = _sc_aggregate(h, pk3, with_counts=False, chunk=40, nbuf=5)
    r2 = _tc_linear_r(h, W_r2)
    return _tc_combine(p2, c1, r2, W_l2, b_l2)
